# baseline (speedup 1.0000x reference)
import functools

import jax
import jax.numpy as jnp
from jax import lax
from jax.experimental import pallas as pl
from jax.experimental.pallas import tpu as pltpu
from jax.experimental.pallas import tpu_sc as plsc

_NC = 2
_NS = 16
_NW = _NC * _NS
_CHUNK = 32


def _make_sc_copy(rows, d, dtype):
    rows_per_w = rows // _NW
    nch = rows_per_w // _CHUNK
    mesh = plsc.VectorSubcoreMesh(core_axis_name="c", subcore_axis_name="s")

    @functools.partial(
        pl.kernel,
        mesh=mesh,
        out_type=jax.ShapeDtypeStruct((rows, d), dtype),
        scratch_types=[
            pltpu.SemaphoreType.DMA,
        ],
    )
    def sc_copy(in_hbm, out_hbm, sem):
        wid = lax.axis_index("s") * _NC + lax.axis_index("c")
        base = wid * rows_per_w
        cp = pltpu.make_async_copy(
            in_hbm.at[pl.ds(base, rows_per_w)],
            out_hbm.at[pl.ds(base, rows_per_w)],
            sem,
        )
        cp.start()
        cp.wait()

    return sc_copy


def kernel(sequences, batch_first, padding_value, padding_side):
    B, L, D = sequences.shape
    rows = B * L
    flat = sequences.reshape(rows, D)
    out = _make_sc_copy(rows, D, sequences.dtype)(flat)
    return out.reshape(B, L, D)


# speedup vs baseline: 31.7337x; 31.7337x over previous
import functools

import jax
import jax.numpy as jnp
from jax import lax
from jax.experimental import pallas as pl
from jax.experimental.pallas import tpu as pltpu
from jax.experimental.pallas import tpu_sc as plsc

_NC = 2
_NS = 16
_NW = _NC * _NS
_CHUNK = 32
_NBUF = 3


def _make_sc_copy(rows, d, dtype):
    rows_per_w = rows // _NW
    nch = rows_per_w // _CHUNK
    mesh = plsc.VectorSubcoreMesh(core_axis_name="c", subcore_axis_name="s")

    nbuf = _NBUF

    @functools.partial(
        pl.kernel,
        mesh=mesh,
        out_type=jax.ShapeDtypeStruct((rows, d), dtype),
        scratch_types=(
            [pltpu.VMEM((_CHUNK, d), dtype) for _ in range(nbuf)]
            + [pltpu.SemaphoreType.DMA for _ in range(2 * nbuf)]
        ),
    )
    def sc_copy(in_hbm, out_hbm, *scratch):
        bufs = scratch[:nbuf]
        rsems = scratch[nbuf:2 * nbuf]
        wsems = scratch[2 * nbuf:]
        wid = lax.axis_index("s") * _NC + lax.axis_index("c")
        base = wid * rows_per_w

        def rd(i):
            return pltpu.make_async_copy(
                in_hbm.at[pl.ds(base + i * _CHUNK, _CHUNK)],
                bufs[i % nbuf], rsems[i % nbuf])

        def wr(i):
            return pltpu.make_async_copy(
                bufs[i % nbuf],
                out_hbm.at[pl.ds(base + i * _CHUNK, _CHUNK)], wsems[i % nbuf])

        for k in range(min(nbuf - 1, nch)):
            rd(k).start()
        for i in range(nch):
            j = i + nbuf - 1
            if j < nch:
                if j - nbuf >= 0:
                    wr(j - nbuf).wait()
                rd(j).start()
            rd(i).wait()
            wr(i).start()
        for k in range(max(0, nch - nbuf), nch):
            wr(k).wait()

    return sc_copy


def kernel(sequences, batch_first, padding_value, padding_side):
    B, L, D = sequences.shape
    rows = B * L
    flat = sequences.reshape(rows, D)
    out = _make_sc_copy(rows, D, sequences.dtype)(flat)
    return out.reshape(B, L, D)
